# baseline (speedup 1.0000x reference)
import functools

import jax
import jax.numpy as jnp
from jax import lax
from jax.experimental import pallas as pl
from jax.experimental.pallas import tpu as pltpu
from jax.experimental.pallas import tpu_sc as plsc

N_NODES = 10000
D = 128
E = 320000
NC = 2
NS = 16
L = 16
NW = NC * NS
E_PER_W = E // NW
CHUNK = 80
STEPS = E_PER_W // CHUNK


def _normalize_rows(z):
    def body(z_ref, op_ref, on_ref):
        x = z_ref[...]
        s = jnp.sum(x * x, axis=1, keepdims=True)
        zh = x * lax.rsqrt(s)
        op_ref[...] = zh
        on_ref[...] = -zh

    return pl.pallas_call(
        body,
        out_shape=[
            jax.ShapeDtypeStruct((N_NODES, D), jnp.float32),
            jax.ShapeDtypeStruct((N_NODES, D), jnp.float32),
        ],
        grid=(10,),
        in_specs=[pl.BlockSpec((N_NODES // 10, D), lambda i: (i, 0))],
        out_specs=[
            pl.BlockSpec((N_NODES // 10, D), lambda i: (i, 0)),
            pl.BlockSpec((N_NODES // 10, D), lambda i: (i, 0)),
        ],
    )(z)


def _rsqrt16(q):
    i = plsc.bitcast(q, jnp.int32)
    i = jnp.int32(0x5F3759DF) - (i >> 1)
    y = plsc.bitcast(i, jnp.float32)
    for _ in range(3):
        y = y * (1.5 - 0.5 * q * y * y)
    return y


def _sc_decode(zpos, zneg, src, dst):
    mesh = plsc.VectorSubcoreMesh(
        core_axis_name="c", subcore_axis_name="s", num_cores=NC, num_subcores=NS
    )

    @functools.partial(
        pl.kernel,
        out_type=jax.ShapeDtypeStruct((E,), jnp.float32),
        mesh=mesh,
        scratch_types=[
            pltpu.VMEM((E_PER_W,), jnp.int32),
            pltpu.VMEM((E_PER_W,), jnp.int32),
            pltpu.VMEM((E_PER_W,), jnp.float32),
            pltpu.VMEM((CHUNK, D), jnp.float32),
            pltpu.VMEM((CHUNK, D), jnp.float32),
            pltpu.VMEM((CHUNK * L,), jnp.float32),
            pltpu.SemaphoreType.DMA,
            pltpu.SemaphoreType.DMA,
            pltpu.SemaphoreType.DMA,
            pltpu.SemaphoreType.DMA,
        ],
        compiler_params=pltpu.CompilerParams(needs_layout_passes=False),
    )
    def k(zp_hbm, zn_hbm, src_hbm, dst_hbm, out_hbm,
          idx_a_all, idx_b_all, out_all, dd0, dd1, tmp,
          sa0, sb0, sa1, sb1):
        wid = lax.axis_index("s") * NC + lax.axis_index("c")
        base = pl.multiple_of(wid * E_PER_W, 16)
        pltpu.sync_copy(src_hbm.at[pl.ds(base, E_PER_W)], idx_a_all)
        pltpu.sync_copy(dst_hbm.at[pl.ds(base, E_PER_W)], idx_b_all)

        bufs = ((dd0, sa0, sb0), (dd1, sa1, sb1))

        def copy_a(s, bi):
            dd, sa, _ = bufs[bi]
            sl = pl.ds(pl.multiple_of(s * CHUNK, 16), CHUNK)
            return pltpu.make_async_copy(zp_hbm.at[idx_a_all.at[sl]], dd, sa)

        def copy_b(s, bi):
            dd, _, sb = bufs[bi]
            sl = pl.ds(pl.multiple_of(s * CHUNK, 16), CHUNK)
            return pltpu.make_async_copy(zn_hbm.at[idx_b_all.at[sl]], dd, sb)

        def issue_a(s, bi):
            copy_a(s, bi).start()

        def drain_a_issue_b(s, bi):
            copy_a(s, bi).wait()
            copy_b(s, bi).start(add=True)

        def compute(s, bi):
            dd = bufs[bi][0]
            copy_b(s, bi).wait()
            obase = s * CHUNK

            @plsc.parallel_loop(0, CHUNK, 1, unroll=8)
            def _(e):
                accs = [jnp.zeros((L,), jnp.float32) for _ in range(4)]
                for kk in range(D // L):
                    v = dd[e, pl.ds(kk * L, L)]
                    accs[kk % 4] = accs[kk % 4] + v * v
                tmp[pl.ds(e * L, L)] = (accs[0] + accs[1]) + (accs[2] + accs[3])

            def group(g, carry):
                q = jnp.zeros((L,), jnp.float32)
                lanes = lax.iota(jnp.int32, L) * L + g * (L * L)
                for l in range(L):
                    q = q + plsc.load_gather(tmp, [lanes + l])
                q = jnp.maximum(q, 1e-30)
                dist = q * _rsqrt16(q)
                out_all[pl.ds(obase + g * L, L)] = 1.0 / (1.0 + jnp.exp(dist - 1.0))
                return carry

            lax.fori_loop(0, CHUNK // L, group, None)

        issue_a(0, 0)
        issue_a(1, 1)
        drain_a_issue_b(0, 0)

        def pair(t, carry):
            s0 = 2 * t
            drain_a_issue_b(s0 + 1, 1)
            compute(s0, 0)
            issue_a(s0 + 2, 0)
            compute(s0 + 1, 1)

            @pl.when(s0 + 3 < STEPS)
            def _():
                issue_a(s0 + 3, 1)

            drain_a_issue_b(s0 + 2, 0)
            return carry

        lax.fori_loop(0, (STEPS - 1) // 2, pair, None)
        compute(STEPS - 1, 0)
        pltpu.sync_copy(out_all, out_hbm.at[pl.ds(base, E_PER_W)])

    return k(zpos, zneg, src, dst)


def kernel(z, edge_index):
    idx = edge_index.astype(jnp.int32)
    zpos, zneg = _normalize_rows(z.astype(jnp.float32))
    return _sc_decode(zpos, zneg, idx[0], idx[1])


# speedup vs baseline: 1.2562x; 1.2562x over previous
import functools

import jax
import jax.numpy as jnp
from jax import lax
from jax.experimental import pallas as pl
from jax.experimental.pallas import tpu as pltpu
from jax.experimental.pallas import tpu_sc as plsc

N_NODES = 10000
D = 128
E = 320000
NC = 2
NS = 16
L = 16
NW = NC * NS
E_PER_W = E // NW
CHUNK = 80
STEPS = E_PER_W // CHUNK


def _normalize_rows(z):
    def body(z_ref, op_ref, on_ref):
        x = z_ref[...]
        s = jnp.sum(x * x, axis=1, keepdims=True)
        zh = x * lax.rsqrt(s)
        op_ref[...] = zh
        on_ref[...] = -zh

    return pl.pallas_call(
        body,
        out_shape=[
            jax.ShapeDtypeStruct((N_NODES, D), jnp.float32),
            jax.ShapeDtypeStruct((N_NODES, D), jnp.float32),
        ],
        grid=(10,),
        in_specs=[pl.BlockSpec((N_NODES // 10, D), lambda i: (i, 0))],
        out_specs=[
            pl.BlockSpec((N_NODES // 10, D), lambda i: (i, 0)),
            pl.BlockSpec((N_NODES // 10, D), lambda i: (i, 0)),
        ],
    )(z)


def _rsqrt16(q):
    i = plsc.bitcast(q, jnp.int32)
    i = jnp.int32(0x5F3759DF) - (i >> 1)
    y = plsc.bitcast(i, jnp.float32)
    for _ in range(3):
        y = y * (1.5 - 0.5 * q * y * y)
    return y


def _sc_decode(zpos, zneg, src, dst):
    mesh = plsc.VectorSubcoreMesh(
        core_axis_name="c", subcore_axis_name="s", num_cores=NC, num_subcores=NS
    )

    @functools.partial(
        pl.kernel,
        out_type=jax.ShapeDtypeStruct((E,), jnp.float32),
        mesh=mesh,
        scratch_types=[
            pltpu.VMEM((E_PER_W,), jnp.int32),
            pltpu.VMEM((E_PER_W,), jnp.int32),
            pltpu.VMEM((E_PER_W,), jnp.float32),
            pltpu.VMEM((CHUNK, D), jnp.float32),
            pltpu.VMEM((CHUNK, D), jnp.float32),
            pltpu.VMEM((CHUNK, D), jnp.float32),
            pltpu.VMEM((CHUNK * L,), jnp.float32),
            pltpu.SemaphoreType.DMA,
            pltpu.SemaphoreType.DMA,
            pltpu.SemaphoreType.DMA,
            pltpu.SemaphoreType.DMA,
            pltpu.SemaphoreType.DMA,
            pltpu.SemaphoreType.DMA,
        ],
        compiler_params=pltpu.CompilerParams(needs_layout_passes=False),
    )
    def k(zp_hbm, zn_hbm, src_hbm, dst_hbm, out_hbm,
          idx_a_all, idx_b_all, out_all, dd0, dd1, dd2, tmp,
          sa0, sb0, sa1, sb1, sa2, sb2):
        wid = lax.axis_index("s") * NC + lax.axis_index("c")
        base = pl.multiple_of(wid * E_PER_W, 16)
        pltpu.sync_copy(src_hbm.at[pl.ds(base, E_PER_W)], idx_a_all)
        pltpu.sync_copy(dst_hbm.at[pl.ds(base, E_PER_W)], idx_b_all)

        bufs = ((dd0, sa0, sb0), (dd1, sa1, sb1), (dd2, sa2, sb2))

        def copy_a(s, bi):
            dd, sa, _ = bufs[bi]
            sl = pl.ds(pl.multiple_of(s * CHUNK, 16), CHUNK)
            return pltpu.make_async_copy(zp_hbm.at[idx_a_all.at[sl]], dd, sa)

        def copy_b(s, bi):
            dd, _, sb = bufs[bi]
            sl = pl.ds(pl.multiple_of(s * CHUNK, 16), CHUNK)
            return pltpu.make_async_copy(zn_hbm.at[idx_b_all.at[sl]], dd, sb)

        def issue_a(s, bi):
            copy_a(s, bi).start()

        def drain_a_issue_b(s, bi):
            copy_a(s, bi).wait()
            copy_b(s, bi).start(add=True)

        def compute(s, bi):
            dd = bufs[bi][0]
            copy_b(s, bi).wait()
            obase = s * CHUNK

            @plsc.parallel_loop(0, CHUNK, 1, unroll=8)
            def _(e):
                accs = [jnp.zeros((L,), jnp.float32) for _ in range(4)]
                for kk in range(D // L):
                    v = dd[e, pl.ds(kk * L, L)]
                    accs[kk % 4] = accs[kk % 4] + v * v
                tmp[pl.ds(e * L, L)] = (accs[0] + accs[1]) + (accs[2] + accs[3])

            def group(g, carry):
                q = jnp.zeros((L,), jnp.float32)
                lanes = lax.iota(jnp.int32, L) * L + g * (L * L)
                for l in range(L):
                    q = q + plsc.load_gather(tmp, [lanes + l])
                q = jnp.maximum(q, 1e-30)
                dist = q * _rsqrt16(q)
                out_all[pl.ds(obase + g * L, L)] = 1.0 / (1.0 + jnp.exp(dist - 1.0))
                return carry

            lax.fori_loop(0, CHUNK // L, group, None)

        issue_a(0, 0)
        issue_a(1, 1)
        issue_a(2, 2)
        drain_a_issue_b(0, 0)
        drain_a_issue_b(1, 1)

        def triple(t, carry):
            for r in range(3):
                s = 3 * t + r
                compute(s, r)

                @pl.when(s + 3 < STEPS)
                def _():
                    issue_a(s + 3, r)

                @pl.when(s + 2 < STEPS)
                def _():
                    drain_a_issue_b(s + 2, (r + 2) % 3)
            return carry

        lax.fori_loop(0, STEPS // 3, triple, None)
        compute(STEPS - 2, 0)
        compute(STEPS - 1, 1)
        pltpu.sync_copy(out_all, out_hbm.at[pl.ds(base, E_PER_W)])

    return k(zpos, zneg, src, dst)


def kernel(z, edge_index):
    idx = edge_index.astype(jnp.int32)
    zpos, zneg = _normalize_rows(z.astype(jnp.float32))
    return _sc_decode(zpos, zneg, idx[0], idx[1])
